# baseline (speedup 1.0000x reference)
import functools

import jax
import jax.numpy as jnp
from jax import lax
from jax.experimental import pallas as pl
from jax.experimental.pallas import tpu as pltpu
from jax.experimental.pallas import tpu_sc as plsc

D_MODEL = 256
D_FF = 1024
N_LAYERS = 4
N_HEADS = 2
DH = D_MODEL // N_HEADS
SCALE = 1.0 / (DH ** 0.5)
N_BINS = 256
LN_EPS = 1e-5

F32 = jnp.float32


def _row_sum(xx):
    S, D = xx.shape
    y = xx.reshape(S, D // 8, 8)
    P = y[:, 0, :]
    for j in range(1, D // 8):
        P = P + y[:, j, :]
    s1 = P[:, :4] + P[:, 4:]
    s2 = s1[:, :2] + s1[:, 2:]
    return s2[:, :1] + s2[:, 1:]


def _layer_norm(x, g, b):
    D = x.shape[-1]
    m = _row_sum(x) * (1.0 / D)
    d = x - m
    v = _row_sum(d * d) * (1.0 / D)
    return g * d / jnp.sqrt(v + LN_EPS) + b


def _decoder_stack(x, wq, wk, wv, wo, l1g, l1b, l2g, l2b, w1, b1, w2, b2):
    S = x.shape[0]
    for l in range(N_LAYERS):
        q = jnp.dot(x, wq[l], preferred_element_type=F32)
        k = jnp.dot(x, wk[l], preferred_element_type=F32)
        v = jnp.dot(x, wv[l], preferred_element_type=F32)
        heads = []
        for h in range(N_HEADS):
            sl = slice(h * DH, (h + 1) * DH)
            qh = q[:, sl]
            kh = k[:, sl]
            vh = v[:, sl]
            s = lax.dot_general(qh, kh, (((1,), (1,)), ((), ())),
                                preferred_element_type=F32) * SCALE
            m = jnp.max(s, axis=-1, keepdims=True)
            e = jnp.exp(s - m)
            den = jnp.sum(e, axis=-1, keepdims=True)
            heads.append(jnp.dot(e, vh, preferred_element_type=F32) / den)
        o = jnp.concatenate(heads, axis=1)
        o = jnp.dot(o, wo[l], preferred_element_type=F32)
        x = _layer_norm(x + o, l1g[l], l1b[l])
        hdn = jnp.maximum(jnp.dot(x, w1[l], preferred_element_type=F32) + b1[l], 0.0)
        hdn = jnp.dot(hdn, w2[l], preferred_element_type=F32) + b2[l]
        x = _layer_norm(x + hdn, l2g[l], l2b[l])
    return x


def _dur_body(x_ref, len_ref, wq, wk, wv, wo, l1g, l1b, l2g, l2b,
              w1, b1, w2, b2, lw, lb, out_ref):
    x = _decoder_stack(x_ref[0], wq, wk, wv, wo, l1g, l1b, l2g, l2b,
                       w1, b1, w2, b2)
    raw = jnp.dot(x, lw[...], preferred_element_type=F32)[:, 0] + lb[0]
    out_ref[0, 0] = (jnp.tanh(raw) + 1.0) * len_ref[0, 0]


def _dur_call(x, lengths3, p):
    B, S, d = x.shape
    full = lambda a: pl.BlockSpec(a.shape, lambda i: (0,) * a.ndim)
    w_args = (p['Wq'], p['Wk'], p['Wv'], p['Wo'], p['ln1_g'], p['ln1_b'],
              p['ln2_g'], p['ln2_b'], p['W1'], p['b1'], p['W2'], p['b2'],
              p['lin_W'], p['lin_b'])
    out = pl.pallas_call(
        _dur_body,
        grid=(B,),
        in_specs=[pl.BlockSpec((1, S, d), lambda i: (i, 0, 0)),
                  pl.BlockSpec((1, 1, S), lambda i: (i, 0, 0))]
                 + [full(a) for a in w_args],
        out_specs=pl.BlockSpec((1, 1, S), lambda i: (i, 0, 0)),
        out_shape=jax.ShapeDtypeStruct((B, 1, S), F32),
    )(x, lengths3, *w_args)
    return out.reshape(B, S)


def _idx_body(dur_ref, pt_ref, et_ref, gidx_ref, pidx_ref, eidx_ref, mel_ref,
              *, B, S, T, sentinel):
    df = dur_ref[...].astype(F32)
    r = lax.broadcasted_iota(jnp.int32, (S, S), 0)
    c = lax.broadcasted_iota(jnp.int32, (S, S), 1)
    tri = (r <= c).astype(F32)
    cs = jnp.dot(df, tri, preferred_element_type=F32)
    melf = jnp.minimum(cs[:, S - 1], float(T))
    mel_ref[...] = melf.astype(jnp.int32)
    t2 = lax.broadcasted_iota(jnp.int32, (T, S), 0).astype(F32)
    t1 = lax.broadcasted_iota(jnp.int32, (T,), 0).astype(F32)
    for b in range(B):
        csb = cs[b, :][None, :]
        cnt = jnp.sum((csb <= t2).astype(F32), axis=1)
        idx = jnp.minimum(cnt, float(S - 1))
        valid = t1 < melf[b]
        g = jnp.where(valid, idx + float(b * S), float(sentinel))
        gidx_ref[pl.ds(b * T, T)] = g.astype(jnp.int32)
    pidx_ref[...] = jnp.clip(jnp.ceil(pt_ref[...] * float(N_BINS)),
                             0.0, float(2 * N_BINS - 1)).astype(jnp.int32)
    eidx_ref[...] = jnp.clip(jnp.ceil(et_ref[...] * float(N_BINS)),
                             0.0, float(2 * N_BINS - 1)).astype(jnp.int32)


def _idx_call(duration, pt_flat, et_flat, B, S, T, sentinel):
    BT = B * T
    body = functools.partial(_idx_body, B=B, S=S, T=T, sentinel=sentinel)
    return pl.pallas_call(
        body,
        out_shape=(jax.ShapeDtypeStruct((BT,), jnp.int32),
                   jax.ShapeDtypeStruct((BT,), jnp.int32),
                   jax.ShapeDtypeStruct((BT,), jnp.int32),
                   jax.ShapeDtypeStruct((B,), jnp.int32)),
    )(duration, pt_flat, et_flat)


_NC, _NS = 2, 16
_NW = _NC * _NS
_SUB = 64


def _sc_gather_body(xz, pemb, eemb, gidx, pidx, eidx,
                    xreg_o, prow_o, erow_o,
                    gidx_v, pidx_v, eidx_v, bufx, bufp, bufe,
                    sem0, sem1, sem2, *, rows_per_w):
    wid = lax.axis_index("s") * _NC + lax.axis_index("c")
    base = wid * rows_per_w
    for sub in range(rows_per_w // _SUB):
        off = base + sub * _SUB
        pltpu.sync_copy(gidx.at[pl.ds(off, _SUB)], gidx_v)
        pltpu.sync_copy(pidx.at[pl.ds(off, _SUB)], pidx_v)
        pltpu.sync_copy(eidx.at[pl.ds(off, _SUB)], eidx_v)
        cx = pltpu.async_copy(xz.at[gidx_v], bufx, sem0)
        cp = pltpu.async_copy(pemb.at[pidx_v], bufp, sem1)
        ce = pltpu.async_copy(eemb.at[eidx_v], bufe, sem2)
        cx.wait()
        pltpu.sync_copy(bufx, xreg_o.at[pl.ds(off, _SUB)])
        cp.wait()
        pltpu.sync_copy(bufp, prow_o.at[pl.ds(off, _SUB)])
        ce.wait()
        pltpu.sync_copy(bufe, erow_o.at[pl.ds(off, _SUB)])


def _sc_gather_call(xz, pemb, eemb, gidx, pidx, eidx, BT, d):
    rows_per_w = BT // _NW
    mesh = plsc.VectorSubcoreMesh(core_axis_name="c", subcore_axis_name="s")
    body = functools.partial(_sc_gather_body, rows_per_w=rows_per_w)
    shp = jax.ShapeDtypeStruct((BT, d), F32)
    f = pl.kernel(
        body,
        out_type=(shp, shp, shp),
        mesh=mesh,
        scratch_types=[pltpu.VMEM((_SUB,), jnp.int32),
                       pltpu.VMEM((_SUB,), jnp.int32),
                       pltpu.VMEM((_SUB,), jnp.int32),
                       pltpu.VMEM((_SUB, d), F32),
                       pltpu.VMEM((_SUB, d), F32),
                       pltpu.VMEM((_SUB, d), F32),
                       pltpu.SemaphoreType.DMA,
                       pltpu.SemaphoreType.DMA,
                       pltpu.SemaphoreType.DMA],
    )
    return f(xz, pemb, eemb, gidx, pidx, eidx)


def _pe_body(x_ref, pr_ref, er_ref, wq, wk, wv, wo, l1g, l1b, l2g, l2b,
             w1, b1, w2, b2, lw, lb, pred_ref, out_ref):
    i = pl.program_id(0)
    xin = x_ref[0]
    out_ref[0] = xin + pr_ref[0] + er_ref[0]
    x = _decoder_stack(xin, wq[0], wk[0], wv[0], wo[0], l1g[0], l1b[0],
                       l2g[0], l2b[0], w1[0], b1[0], w2[0], b2[0])
    raw = jnp.dot(x, lw[0], preferred_element_type=F32)[:, 0] + lb[0, 0, 0]
    pred_ref[0, 0, 0] = jnp.where(i < 4, jnp.maximum(raw, 0.0), raw)


def _pe_call(xreg, prow, erow, sp):
    B, T, d = xreg.shape
    G = 2 * B
    bs_x = pl.BlockSpec((1, T, d), lambda i: (i % B, 0, 0))

    def wspec(a):
        return pl.BlockSpec((1,) + a.shape[1:],
                            lambda i, nd=a.ndim: (i // B,) + (0,) * (nd - 1))

    w_args = (sp['Wq'], sp['Wk'], sp['Wv'], sp['Wo'], sp['ln1_g'], sp['ln1_b'],
              sp['ln2_g'], sp['ln2_b'], sp['W1'], sp['b1'], sp['W2'], sp['b2'],
              sp['lin_W'], sp['lin_b'])
    pred, out = pl.pallas_call(
        _pe_body,
        grid=(G,),
        in_specs=[bs_x, bs_x, bs_x] + [wspec(a) for a in w_args],
        out_specs=(pl.BlockSpec((1, 1, 1, T), lambda i: (i // B, i % B, 0, 0)),
                   pl.BlockSpec((1, T, d), lambda i: (i % B, 0, 0))),
        out_shape=(jax.ShapeDtypeStruct((2, B, 1, T), F32),
                   jax.ShapeDtypeStruct((B, T, d), F32)),
    )(xreg, prow, erow, *w_args)
    return pred, out


def kernel(x, src_seq, src_mask, mel_mask, duration_target, pitch_target,
           energy_target, max_len, params):
    B, S, d = x.shape
    T = mel_mask.shape[1]
    BT = B * T

    lengths3 = src_seq[:, :, 2].reshape(B, 1, S)
    log_dur = _dur_call(x, lengths3, params['dur'])

    sentinel = B * S
    gidx, pidx, eidx, mel_len = _idx_call(
        duration_target, pitch_target.reshape(BT), energy_target.reshape(BT),
        B, S, T, sentinel)

    xz = jnp.concatenate(
        [x.reshape(B * S, d), jnp.zeros((8, d), F32)], axis=0)
    xreg_f, prow_f, erow_f = _sc_gather_call(
        xz, params['pitch_emb'], params['energy_emb'], gidx, pidx, eidx, BT, d)

    sp = {k: jnp.stack([params['pit'][k], params['eng'][k]])
          for k in params['pit']}
    sp['lin_b'] = sp['lin_b'].reshape(2, 1, 1)
    pred, out = _pe_call(xreg_f.reshape(B, T, d), prow_f.reshape(B, T, d),
                         erow_f.reshape(B, T, d), sp)
    pitch_prediction = pred[0, :, 0, :]
    energy_prediction = pred[1, :, 0, :]
    return (out, log_dur, pitch_prediction, energy_prediction,
            mel_len, mel_mask)


# speedup vs baseline: 8.4607x; 8.4607x over previous
import functools

import jax
import jax.numpy as jnp
from jax import lax
from jax.experimental import pallas as pl
from jax.experimental.pallas import tpu as pltpu
from jax.experimental.pallas import tpu_sc as plsc

D_MODEL = 256
D_FF = 1024
N_LAYERS = 4
N_HEADS = 2
DH = D_MODEL // N_HEADS
SCALE = 1.0 / (DH ** 0.5)
N_BINS = 256
LN_EPS = 1e-5

F32 = jnp.float32


def _layer_norm(x, g, b):
    m = jnp.mean(x, axis=-1, keepdims=True)
    v = jnp.mean((x - m) * (x - m), axis=-1, keepdims=True)
    return g * (x - m) / jnp.sqrt(v + LN_EPS) + b


def _decoder_stack(x, wq, wk, wv, wo, l1g, l1b, l2g, l2b, w1, b1, w2, b2):
    S = x.shape[0]
    for l in range(N_LAYERS):
        q = jnp.dot(x, wq[l], preferred_element_type=F32)
        k = jnp.dot(x, wk[l], preferred_element_type=F32)
        v = jnp.dot(x, wv[l], preferred_element_type=F32)
        heads = []
        for h in range(N_HEADS):
            sl = slice(h * DH, (h + 1) * DH)
            qh = q[:, sl]
            kh = k[:, sl]
            vh = v[:, sl]
            s = lax.dot_general(qh, kh, (((1,), (1,)), ((), ())),
                                preferred_element_type=F32) * SCALE
            m = jnp.max(s, axis=-1, keepdims=True)
            e = jnp.exp(s - m)
            den = jnp.sum(e, axis=-1, keepdims=True)
            heads.append(jnp.dot(e, vh, preferred_element_type=F32) / den)
        o = jnp.concatenate(heads, axis=1)
        o = jnp.dot(o, wo[l], preferred_element_type=F32)
        x = _layer_norm(x + o, l1g[l], l1b[l])
        hdn = jnp.maximum(jnp.dot(x, w1[l], preferred_element_type=F32) + b1[l], 0.0)
        hdn = jnp.dot(hdn, w2[l], preferred_element_type=F32) + b2[l]
        x = _layer_norm(x + hdn, l2g[l], l2b[l])
    return x


def _dur_body(x_ref, len_ref, wq, wk, wv, wo, l1g, l1b, l2g, l2b,
              w1, b1, w2, b2, lw, lb, out_ref):
    x = _decoder_stack(x_ref[0], wq, wk, wv, wo, l1g, l1b, l2g, l2b,
                       w1, b1, w2, b2)
    raw = jnp.dot(x, lw[...], preferred_element_type=F32)[:, 0] + lb[0]
    out_ref[0, 0] = (jnp.tanh(raw) + 1.0) * len_ref[0, 0]


def _dur_call(x, lengths3, p):
    B, S, d = x.shape
    full = lambda a: pl.BlockSpec(a.shape, lambda i: (0,) * a.ndim)
    w_args = (p['Wq'], p['Wk'], p['Wv'], p['Wo'], p['ln1_g'], p['ln1_b'],
              p['ln2_g'], p['ln2_b'], p['W1'], p['b1'], p['W2'], p['b2'],
              p['lin_W'], p['lin_b'])
    out = pl.pallas_call(
        _dur_body,
        grid=(B,),
        in_specs=[pl.BlockSpec((1, S, d), lambda i: (i, 0, 0)),
                  pl.BlockSpec((1, 1, S), lambda i: (i, 0, 0))]
                 + [full(a) for a in w_args],
        out_specs=pl.BlockSpec((1, 1, S), lambda i: (i, 0, 0)),
        out_shape=jax.ShapeDtypeStruct((B, 1, S), F32),
    )(x, lengths3, *w_args)
    return out.reshape(B, S)


def _idx_body(dur_ref, pt_ref, et_ref, gidx_ref, pidx_ref, eidx_ref, mel_ref,
              *, B, S, T, sentinel):
    df = dur_ref[...].astype(F32)
    r = lax.broadcasted_iota(jnp.int32, (S, S), 0)
    c = lax.broadcasted_iota(jnp.int32, (S, S), 1)
    tri = (r <= c).astype(F32)
    cs = jnp.dot(df, tri, preferred_element_type=F32)
    melf = jnp.minimum(cs[:, S - 1], float(T))
    mel_ref[...] = melf.astype(jnp.int32)
    t2 = lax.broadcasted_iota(jnp.int32, (T, S), 0).astype(F32)
    t1 = lax.broadcasted_iota(jnp.int32, (T,), 0).astype(F32)
    for b in range(B):
        csb = cs[b, :][None, :]
        cnt = jnp.sum((csb <= t2).astype(F32), axis=1)
        idx = jnp.minimum(cnt, float(S - 1))
        valid = t1 < melf[b]
        g = jnp.where(valid, idx + float(b * S), float(sentinel))
        gidx_ref[pl.ds(b * T, T)] = g.astype(jnp.int32)
    pidx_ref[...] = jnp.clip(jnp.ceil(pt_ref[...] * float(N_BINS)),
                             0.0, float(2 * N_BINS - 1)).astype(jnp.int32)
    eidx_ref[...] = jnp.clip(jnp.ceil(et_ref[...] * float(N_BINS)),
                             0.0, float(2 * N_BINS - 1)).astype(jnp.int32)


def _idx_call(duration, pt_flat, et_flat, B, S, T, sentinel):
    BT = B * T
    body = functools.partial(_idx_body, B=B, S=S, T=T, sentinel=sentinel)
    return pl.pallas_call(
        body,
        out_shape=(jax.ShapeDtypeStruct((BT,), jnp.int32),
                   jax.ShapeDtypeStruct((BT,), jnp.int32),
                   jax.ShapeDtypeStruct((BT,), jnp.int32),
                   jax.ShapeDtypeStruct((B,), jnp.int32)),
    )(duration, pt_flat, et_flat)


_NC, _NS = 2, 16
_NW = _NC * _NS
_SUB = 64


def _sc_gather_body(xz, pemb, eemb, gidx, pidx, eidx,
                    xreg_o, prow_o, erow_o,
                    gidx_v, pidx_v, eidx_v, bufx, bufp, bufe,
                    sem0, sem1, sem2, *, rows_per_w):
    wid = lax.axis_index("s") * _NC + lax.axis_index("c")
    base = wid * rows_per_w
    for sub in range(rows_per_w // _SUB):
        off = base + sub * _SUB
        pltpu.sync_copy(gidx.at[pl.ds(off, _SUB)], gidx_v)
        pltpu.sync_copy(pidx.at[pl.ds(off, _SUB)], pidx_v)
        pltpu.sync_copy(eidx.at[pl.ds(off, _SUB)], eidx_v)
        cx = pltpu.async_copy(xz.at[gidx_v], bufx, sem0)
        cp = pltpu.async_copy(pemb.at[pidx_v], bufp, sem1)
        ce = pltpu.async_copy(eemb.at[eidx_v], bufe, sem2)
        cx.wait()
        pltpu.sync_copy(bufx, xreg_o.at[pl.ds(off, _SUB)])
        cp.wait()
        pltpu.sync_copy(bufp, prow_o.at[pl.ds(off, _SUB)])
        ce.wait()
        pltpu.sync_copy(bufe, erow_o.at[pl.ds(off, _SUB)])


def _sc_gather_call(xz, pemb, eemb, gidx, pidx, eidx, BT, d):
    rows_per_w = BT // _NW
    mesh = plsc.VectorSubcoreMesh(core_axis_name="c", subcore_axis_name="s")
    body = functools.partial(_sc_gather_body, rows_per_w=rows_per_w)
    shp = jax.ShapeDtypeStruct((BT, d), F32)
    f = pl.kernel(
        body,
        out_type=(shp, shp, shp),
        mesh=mesh,
        scratch_types=[pltpu.VMEM((_SUB,), jnp.int32),
                       pltpu.VMEM((_SUB,), jnp.int32),
                       pltpu.VMEM((_SUB,), jnp.int32),
                       pltpu.VMEM((_SUB, d), F32),
                       pltpu.VMEM((_SUB, d), F32),
                       pltpu.VMEM((_SUB, d), F32),
                       pltpu.SemaphoreType.DMA,
                       pltpu.SemaphoreType.DMA,
                       pltpu.SemaphoreType.DMA],
    )
    return f(xz, pemb, eemb, gidx, pidx, eidx)


def _pe_body(x_ref, pr_ref, er_ref, wq, wk, wv, wo, l1g, l1b, l2g, l2b,
             w1, b1, w2, b2, lw, lb, pred_ref, out_ref):
    i = pl.program_id(0)
    xin = x_ref[0]
    out_ref[0] = xin + pr_ref[0] + er_ref[0]
    x = _decoder_stack(xin, wq[0], wk[0], wv[0], wo[0], l1g[0], l1b[0],
                       l2g[0], l2b[0], w1[0], b1[0], w2[0], b2[0])
    raw = jnp.dot(x, lw[0], preferred_element_type=F32)[:, 0] + lb[0, 0, 0]
    pred_ref[0, 0, 0] = jnp.where(i < 4, jnp.maximum(raw, 0.0), raw)


def _pe_call(xreg, prow, erow, sp):
    B, T, d = xreg.shape
    G = 2 * B
    bs_x = pl.BlockSpec((1, T, d), lambda i: (i % B, 0, 0))

    def wspec(a):
        return pl.BlockSpec((1,) + a.shape[1:],
                            lambda i, nd=a.ndim: (i // B,) + (0,) * (nd - 1))

    w_args = (sp['Wq'], sp['Wk'], sp['Wv'], sp['Wo'], sp['ln1_g'], sp['ln1_b'],
              sp['ln2_g'], sp['ln2_b'], sp['W1'], sp['b1'], sp['W2'], sp['b2'],
              sp['lin_W'], sp['lin_b'])
    pred, out = pl.pallas_call(
        _pe_body,
        grid=(G,),
        in_specs=[bs_x, bs_x, bs_x] + [wspec(a) for a in w_args],
        out_specs=(pl.BlockSpec((1, 1, 1, T), lambda i: (i // B, i % B, 0, 0)),
                   pl.BlockSpec((1, T, d), lambda i: (i % B, 0, 0))),
        out_shape=(jax.ShapeDtypeStruct((2, B, 1, T), F32),
                   jax.ShapeDtypeStruct((B, T, d), F32)),
    )(xreg, prow, erow, *w_args)
    return pred, out


def kernel(x, src_seq, src_mask, mel_mask, duration_target, pitch_target,
           energy_target, max_len, params):
    B, S, d = x.shape
    T = mel_mask.shape[1]
    BT = B * T

    lengths3 = src_seq[:, :, 2].reshape(B, 1, S)
    log_dur = _dur_call(x, lengths3, params['dur'])

    sentinel = B * S
    gidx, pidx, eidx, mel_len = _idx_call(
        duration_target, pitch_target.reshape(BT), energy_target.reshape(BT),
        B, S, T, sentinel)

    xz = jnp.concatenate(
        [x.reshape(B * S, d), jnp.zeros((8, d), F32)], axis=0)
    xreg_f, prow_f, erow_f = _sc_gather_call(
        xz, params['pitch_emb'], params['energy_emb'], gidx, pidx, eidx, BT, d)

    sp = {k: jnp.stack([params['pit'][k], params['eng'][k]])
          for k in params['pit']}
    sp['lin_b'] = sp['lin_b'].reshape(2, 1, 1)
    pred, out = _pe_call(xreg_f.reshape(B, T, d), prow_f.reshape(B, T, d),
                         erow_f.reshape(B, T, d), sp)
    pitch_prediction = pred[0, :, 0, :]
    energy_prediction = pred[1, :, 0, :]
    return (out, log_dur, pitch_prediction, energy_prediction,
            mel_len, mel_mask)
